# baseline (speedup 1.0000x reference)
import functools

import jax
import jax.numpy as jnp
from jax import lax
from jax.experimental import pallas as pl
from jax.experimental.pallas import tpu as pltpu
from jax.experimental.pallas import tpu_sc as plsc

_N = 16384
_D = 256
_INV_T = 10.0
_K_TOP = 6553
_K_BOT = 9832

_NW = 32
_AP = 6656
_WROWS = _AP // _NW

_R = 832
_C = 512
_NR = _AP // _R
_NC = _N // _C


def _thresh_body(s_ref, out_ref):
    bits = lax.bitcast_convert_type(s_ref[...], jnp.int32)

    def step(_, carry):
        lo1, hi1, lo2, hi2 = carry

        def upd(lo, hi, k):
            mid = (lo + hi + 1) >> 1
            cnt = jnp.sum((bits >= mid).astype(jnp.int32))
            take = cnt >= k
            return jnp.where(take, mid, lo), jnp.where(take, hi, mid - 1)

        lo1, hi1 = upd(lo1, hi1, _K_TOP)
        lo2, hi2 = upd(lo2, hi2, _K_BOT)
        return lo1, hi1, lo2, hi2

    lo1, _, lo2, _ = lax.fori_loop(
        0, 30, step, (0, 0x3F800000, 0, 0x3F800000))
    out_ref[0] = lax.bitcast_convert_type(lo1, jnp.float32)
    out_ref[1] = lax.bitcast_convert_type(lo2, jnp.float32)


def _sc_body(proj_hbm, scores_hbm, th_hbm, anchors_hbm, cnt_hbm,
             s_v, th_v, idx_v, rows_a, rows_b, cnt_v, sem_a, sem_b):
    wid = lax.axis_index("s") * 2 + lax.axis_index("c")
    base = wid * _WROWS

    pltpu.sync_copy(scores_hbm, s_v)
    pltpu.sync_copy(th_hbm, th_v)
    lane = lax.iota(jnp.int32, 16)
    t_bot = th_v[...][1]

    zero16 = jnp.zeros((16,), jnp.int32)
    for k in range(_WROWS // 16):
        idx_v[pl.ds(base + k * 16, 16)] = zero16

    def body(it, off):
        s = s_v[pl.ds(it * 16, 16)]
        sel = s <= t_bot
        idxvec = lane + it * 16
        plsc.store_compressed(idx_v.at[pl.ds(off, 16)], idxvec, mask=sel)
        return off + jnp.sum(jnp.where(sel, 1, 0))

    total = lax.fori_loop(0, _N // 16, body, 0)

    @pl.when(wid == 0)
    def _write_count():
        cnt_v[...] = zero16 + total
        pltpu.sync_copy(cnt_v, cnt_hbm)

    h = _WROWS // 2
    cp_a = pltpu.async_copy(
        proj_hbm.at[idx_v.at[pl.ds(base, h)]], rows_a, sem_a)
    cp_b = pltpu.async_copy(
        proj_hbm.at[idx_v.at[pl.ds(base + h, h)]], rows_b, sem_b)
    cp_a.wait()
    pltpu.sync_copy(rows_a, anchors_hbm.at[pl.ds(base, h)])
    cp_b.wait()
    pltpu.sync_copy(rows_b, anchors_hbm.at[pl.ds(base + h, h)])


def _main_body(th_ref, a_ref, b_ref, scol_ref, out_ref):
    j = pl.program_id(1)
    t_top = th_ref[0]

    ab = a_ref[...].astype(jnp.bfloat16)
    bb = b_ref[...].astype(jnp.bfloat16)
    s = lax.dot_general(ab, bb, (((1,), (1,)), ((), ())),
                        preferred_element_type=jnp.float32)
    e = jnp.exp(s * _INV_T - _INV_T).astype(jnp.bfloat16)
    m = (scol_ref[0, 0] >= t_top).astype(jnp.float32)
    red = jnp.concatenate(
        [m[:, None], jnp.ones((_C, 1), jnp.float32)], axis=1)
    p = lax.dot_general(e, red.astype(jnp.bfloat16),
                        (((1,), (0,)), ((), ())),
                        preferred_element_type=jnp.float32)
    out_ref[...] = jnp.where(j == 0, p, out_ref[...] + p)


def _fin_body(cnt_ref, acc_ref, a_ref, out_ref):
    cnt = jnp.minimum(cnt_ref[0], _AP)
    a32 = a_ref[...]
    rn = jnp.sum(a32 * a32, axis=1, keepdims=True) * _INV_T
    self_e = jnp.exp(rn - _INV_T)
    accn = acc_ref[:, 0:1]
    accd = acc_ref[:, 1:2] - self_e
    row_ids = lax.broadcasted_iota(jnp.int32, (_AP, 1), 0)
    rm = (row_ids < cnt).astype(jnp.float32)
    vals = (jnp.log(accd) - jnp.log(accn)) * rm
    out_ref[0] = jnp.sum(vals) / cnt.astype(jnp.float32)


_sc_gather = pl.kernel(
    _sc_body,
    out_type=[
        jax.ShapeDtypeStruct((_AP, _D), jnp.float32),
        jax.ShapeDtypeStruct((16,), jnp.int32),
    ],
    mesh=plsc.VectorSubcoreMesh(core_axis_name="c", subcore_axis_name="s",
                                num_cores=2, num_subcores=16),
    scratch_types=[
        pltpu.VMEM((_N,), jnp.float32),
        pltpu.VMEM((16,), jnp.float32),
        pltpu.VMEM((_AP + 16,), jnp.int32),
        pltpu.VMEM((_WROWS // 2, _D), jnp.float32),
        pltpu.VMEM((_WROWS // 2, _D), jnp.float32),
        pltpu.VMEM((16,), jnp.int32),
        pltpu.SemaphoreType.DMA,
        pltpu.SemaphoreType.DMA,
    ],
    compiler_params=pltpu.CompilerParams(needs_layout_passes=False),
)


@jax.jit
def kernel(projections, scores):
    th = pl.pallas_call(
        _thresh_body,
        out_specs=pl.BlockSpec(memory_space=pltpu.SMEM),
        out_shape=jax.ShapeDtypeStruct((2,), jnp.float32),
    )(scores.reshape(128, 128))

    anchors, cnt = _sc_gather(projections, scores, jnp.pad(th, (0, 14)))

    acc = pl.pallas_call(
        _main_body,
        grid=(_NR, _NC),
        in_specs=[
            pl.BlockSpec(memory_space=pltpu.SMEM),
            pl.BlockSpec((_R, _D), lambda i, j: (i, 0)),
            pl.BlockSpec((_C, _D), lambda i, j: (j, 0)),
            pl.BlockSpec((1, 1, _C), lambda i, j: (j, 0, 0)),
        ],
        out_specs=pl.BlockSpec((_R, 2), lambda i, j: (i, 0)),
        out_shape=jax.ShapeDtypeStruct((_AP, 2), jnp.float32),
        compiler_params=pltpu.CompilerParams(
            dimension_semantics=("arbitrary", "arbitrary")),
    )(
        th,
        anchors,
        projections,
        scores.reshape(_NC, 1, _C),
    )

    loss = pl.pallas_call(
        _fin_body,
        in_specs=[
            pl.BlockSpec(memory_space=pltpu.SMEM),
            pl.BlockSpec((_AP, 2), lambda: (0, 0)),
            pl.BlockSpec((_AP, _D), lambda: (0, 0)),
        ],
        out_specs=pl.BlockSpec(memory_space=pltpu.SMEM),
        out_shape=jax.ShapeDtypeStruct((1,), jnp.float32),
    )(cnt, acc, anchors)
    return loss[0]


# speedup vs baseline: 1.3079x; 1.3079x over previous
import functools

import jax
import jax.numpy as jnp
from jax import lax
from jax.experimental import pallas as pl
from jax.experimental.pallas import tpu as pltpu
from jax.experimental.pallas import tpu_sc as plsc

_N = 16384
_D = 256
_INV_T = 10.0
_K_TOP = 6553
_K_BOT = 9832

_NW = 32
_AP = 6656
_WROWS = _AP // _NW

_R = 832
_C = 1024
_NR = _AP // _R
_NC = _N // _C


def _thresh_body(s_ref, out_ref):
    bits = lax.bitcast_convert_type(s_ref[...], jnp.int32)

    def step(_, carry):
        lo1, hi1, lo2, hi2 = carry

        def upd(lo, hi, k):
            mid = (lo + hi + 1) >> 1
            cnt = jnp.sum((bits >= mid).astype(jnp.int32))
            take = cnt >= k
            return jnp.where(take, mid, lo), jnp.where(take, hi, mid - 1)

        lo1, hi1 = upd(lo1, hi1, _K_TOP)
        lo2, hi2 = upd(lo2, hi2, _K_BOT)
        return lo1, hi1, lo2, hi2

    lo1, _, lo2, _ = lax.fori_loop(
        0, 30, step, (0, 0x3F800000, 0, 0x3F800000))
    out_ref[0] = lax.bitcast_convert_type(lo1, jnp.float32)
    out_ref[1] = lax.bitcast_convert_type(lo2, jnp.float32)


def _sc_body(proj_hbm, scores_hbm, th_hbm, anchors_hbm, cnt_hbm,
             s_v, th_v, idx_v, rows_a, rows_b, cnt_v, sem_a, sem_b):
    wid = lax.axis_index("s") * 2 + lax.axis_index("c")
    base = wid * _WROWS

    pltpu.sync_copy(scores_hbm, s_v)
    pltpu.sync_copy(th_hbm, th_v)
    lane = lax.iota(jnp.int32, 16)
    t_bot = th_v[...][1]

    zero16 = jnp.zeros((16,), jnp.int32)
    for k in range(_WROWS // 16):
        idx_v[pl.ds(base + k * 16, 16)] = zero16

    def body(it, off):
        s = s_v[pl.ds(it * 16, 16)]
        sel = s <= t_bot
        idxvec = lane + it * 16
        plsc.store_compressed(idx_v.at[pl.ds(off, 16)], idxvec, mask=sel)
        return off + jnp.sum(jnp.where(sel, 1, 0))

    total = lax.fori_loop(0, _N // 16, body, 0)

    @pl.when(wid == 0)
    def _write_count():
        cnt_v[...] = zero16 + total
        pltpu.sync_copy(cnt_v, cnt_hbm)

    h = _WROWS // 2
    cp_a = pltpu.async_copy(
        proj_hbm.at[idx_v.at[pl.ds(base, h)]], rows_a, sem_a)
    cp_b = pltpu.async_copy(
        proj_hbm.at[idx_v.at[pl.ds(base + h, h)]], rows_b, sem_b)
    cp_a.wait()
    pltpu.sync_copy(rows_a, anchors_hbm.at[pl.ds(base, h)])
    cp_b.wait()
    pltpu.sync_copy(rows_b, anchors_hbm.at[pl.ds(base + h, h)])


def _main_body(th_ref, a_ref, b_ref, scol_ref, out_ref):
    j = pl.program_id(1)
    t_top = th_ref[0]

    ab = a_ref[...].astype(jnp.bfloat16)
    bb = b_ref[...].astype(jnp.bfloat16)
    s = lax.dot_general(ab, bb, (((1,), (1,)), ((), ())),
                        preferred_element_type=jnp.float32)
    e = jnp.exp(s * _INV_T - _INV_T).astype(jnp.bfloat16)
    m = (scol_ref[0, 0] >= t_top).astype(jnp.float32)
    red = jnp.concatenate(
        [m[:, None], jnp.ones((_C, 1), jnp.float32)], axis=1)
    p = lax.dot_general(e, red.astype(jnp.bfloat16),
                        (((1,), (0,)), ((), ())),
                        preferred_element_type=jnp.float32)
    out_ref[...] = jnp.where(j == 0, p, out_ref[...] + p)


def _fin_body(cnt_ref, acc_ref, a_ref, out_ref):
    cnt = jnp.minimum(cnt_ref[0], _AP)
    a32 = a_ref[...]
    rn = jnp.sum(a32 * a32, axis=1, keepdims=True) * _INV_T
    self_e = jnp.exp(rn - _INV_T)
    accn = acc_ref[:, 0:1]
    accd = acc_ref[:, 1:2] - self_e
    row_ids = lax.broadcasted_iota(jnp.int32, (_AP, 1), 0)
    rm = (row_ids < cnt).astype(jnp.float32)
    vals = (jnp.log(accd) - jnp.log(accn)) * rm
    out_ref[0] = jnp.sum(vals) / cnt.astype(jnp.float32)


_sc_gather = pl.kernel(
    _sc_body,
    out_type=[
        jax.ShapeDtypeStruct((_AP, _D), jnp.float32),
        jax.ShapeDtypeStruct((16,), jnp.int32),
    ],
    mesh=plsc.VectorSubcoreMesh(core_axis_name="c", subcore_axis_name="s",
                                num_cores=2, num_subcores=16),
    scratch_types=[
        pltpu.VMEM((_N,), jnp.float32),
        pltpu.VMEM((16,), jnp.float32),
        pltpu.VMEM((_AP + 16,), jnp.int32),
        pltpu.VMEM((_WROWS // 2, _D), jnp.float32),
        pltpu.VMEM((_WROWS // 2, _D), jnp.float32),
        pltpu.VMEM((16,), jnp.int32),
        pltpu.SemaphoreType.DMA,
        pltpu.SemaphoreType.DMA,
    ],
    compiler_params=pltpu.CompilerParams(needs_layout_passes=False),
)


@jax.jit
def kernel(projections, scores):
    th = pl.pallas_call(
        _thresh_body,
        out_specs=pl.BlockSpec(memory_space=pltpu.SMEM),
        out_shape=jax.ShapeDtypeStruct((2,), jnp.float32),
    )(scores.reshape(128, 128))

    anchors, cnt = _sc_gather(projections, scores, jnp.pad(th, (0, 14)))

    acc = pl.pallas_call(
        _main_body,
        grid=(_NR, _NC),
        in_specs=[
            pl.BlockSpec(memory_space=pltpu.SMEM),
            pl.BlockSpec((_R, _D), lambda i, j: (i, 0)),
            pl.BlockSpec((_C, _D), lambda i, j: (j, 0)),
            pl.BlockSpec((1, 1, _C), lambda i, j: (j, 0, 0)),
        ],
        out_specs=pl.BlockSpec((_R, 2), lambda i, j: (i, 0)),
        out_shape=jax.ShapeDtypeStruct((_AP, 2), jnp.float32),
        compiler_params=pltpu.CompilerParams(
            dimension_semantics=("arbitrary", "arbitrary")),
    )(
        th,
        anchors,
        projections,
        scores.reshape(_NC, 1, _C),
    )

    loss = pl.pallas_call(
        _fin_body,
        in_specs=[
            pl.BlockSpec(memory_space=pltpu.SMEM),
            pl.BlockSpec((_AP, 2), lambda: (0, 0)),
            pl.BlockSpec((_AP, _D), lambda: (0, 0)),
        ],
        out_specs=pl.BlockSpec(memory_space=pltpu.SMEM),
        out_shape=jax.ShapeDtypeStruct((1,), jnp.float32),
    )(cnt, acc, anchors)
    return loss[0]


# speedup vs baseline: 1.5775x; 1.2062x over previous
import functools

import jax
import jax.numpy as jnp
from jax import lax
from jax.experimental import pallas as pl
from jax.experimental.pallas import tpu as pltpu
from jax.experimental.pallas import tpu_sc as plsc

_N = 16384
_D = 256
_INV_T = 10.0
_K_TOP = 6553
_K_BOT = 9832

_NW = 32
_AP = 6656
_WROWS = _AP // _NW

_R = 1664
_C = 1024
_NR = _AP // _R
_NC = _N // _C

_C1 = 14.426950408889634


def _cast_scale_body(x_ref, o_ref):
    o_ref[...] = (x_ref[...] * _C1).astype(jnp.bfloat16)


def _cast_body(x_ref, o_ref):
    o_ref[...] = x_ref[...].astype(jnp.bfloat16)


def _thresh_body(s_ref, out_ref):
    bits = lax.bitcast_convert_type(s_ref[...], jnp.int32)

    def step(_, carry):
        lo1, hi1, lo2, hi2 = carry

        def upd(lo, hi, k):
            mid = (lo + hi + 1) >> 1
            cnt = jnp.sum((bits >= mid).astype(jnp.int32))
            take = cnt >= k
            return jnp.where(take, mid, lo), jnp.where(take, hi, mid - 1)

        lo1, hi1 = upd(lo1, hi1, _K_TOP)
        lo2, hi2 = upd(lo2, hi2, _K_BOT)
        return lo1, hi1, lo2, hi2

    lo1, _, lo2, _ = lax.fori_loop(
        0, 30, step, (0, 0x3F800000, 0, 0x3F800000))
    out_ref[0] = lax.bitcast_convert_type(lo1, jnp.float32)
    out_ref[1] = lax.bitcast_convert_type(lo2, jnp.float32)


def _sc_body(proj_hbm, scores_hbm, th_hbm, anchors_hbm, cnt_hbm,
             s_v, th_v, idx_v, rows_a, rows_b, cnt_v, sem_a, sem_b):
    wid = lax.axis_index("s") * 2 + lax.axis_index("c")
    base = wid * _WROWS

    pltpu.sync_copy(scores_hbm, s_v)
    pltpu.sync_copy(th_hbm, th_v)
    lane = lax.iota(jnp.int32, 16)
    t_bot = th_v[...][1]

    zero16 = jnp.zeros((16,), jnp.int32)
    for k in range(_WROWS // 16):
        idx_v[pl.ds(base + k * 16, 16)] = zero16

    def body(it, off):
        s = s_v[pl.ds(it * 16, 16)]
        sel = s <= t_bot
        idxvec = lane + it * 16
        plsc.store_compressed(idx_v.at[pl.ds(off, 16)], idxvec, mask=sel)
        return off + jnp.sum(jnp.where(sel, 1, 0))

    total = lax.fori_loop(0, _N // 16, body, 0)

    @pl.when(wid == 0)
    def _write_count():
        cnt_v[...] = zero16 + total
        pltpu.sync_copy(cnt_v, cnt_hbm)

    h = _WROWS // 2
    cp_a = pltpu.async_copy(
        proj_hbm.at[idx_v.at[pl.ds(base, h)]], rows_a, sem_a)
    cp_b = pltpu.async_copy(
        proj_hbm.at[idx_v.at[pl.ds(base + h, h)]], rows_b, sem_b)
    cp_a.wait()
    pltpu.sync_copy(rows_a, anchors_hbm.at[pl.ds(base, h)])
    cp_b.wait()
    pltpu.sync_copy(rows_b, anchors_hbm.at[pl.ds(base + h, h)])


def _main_body(th_ref, a_ref, b_ref, scol_ref, out_ref):
    j = pl.program_id(1)
    t_top = th_ref[0]

    s = lax.dot_general(a_ref[...], b_ref[...], (((1,), (1,)), ((), ())),
                        preferred_element_type=jnp.float32)
    e = jnp.exp2(s - _C1).astype(jnp.bfloat16)
    m = (scol_ref[0, 0] >= t_top).astype(jnp.float32)
    red = jnp.concatenate(
        [m[:, None], jnp.ones((_C, 1), jnp.float32)], axis=1)
    p = lax.dot_general(e, red.astype(jnp.bfloat16),
                        (((1,), (0,)), ((), ())),
                        preferred_element_type=jnp.float32)
    out_ref[...] = jnp.where(j == 0, p, out_ref[...] + p)


def _fin_body(cnt_ref, acc_ref, a_ref, out_ref):
    cnt = jnp.minimum(cnt_ref[0], _AP)
    a32 = a_ref[...]
    rn = jnp.sum(a32 * a32, axis=1, keepdims=True) * _INV_T
    self_e = jnp.exp(rn - _INV_T)
    accn = acc_ref[:, 0:1]
    accd = acc_ref[:, 1:2] - self_e
    row_ids = lax.broadcasted_iota(jnp.int32, (_AP, 1), 0)
    rm = (row_ids < cnt).astype(jnp.float32)
    vals = (jnp.log(accd) - jnp.log(accn)) * rm
    out_ref[0] = jnp.sum(vals) / cnt.astype(jnp.float32)


_sc_gather = pl.kernel(
    _sc_body,
    out_type=[
        jax.ShapeDtypeStruct((_AP, _D), jnp.float32),
        jax.ShapeDtypeStruct((16,), jnp.int32),
    ],
    mesh=plsc.VectorSubcoreMesh(core_axis_name="c", subcore_axis_name="s",
                                num_cores=2, num_subcores=16),
    scratch_types=[
        pltpu.VMEM((_N,), jnp.float32),
        pltpu.VMEM((16,), jnp.float32),
        pltpu.VMEM((_AP + 16,), jnp.int32),
        pltpu.VMEM((_WROWS // 2, _D), jnp.float32),
        pltpu.VMEM((_WROWS // 2, _D), jnp.float32),
        pltpu.VMEM((16,), jnp.int32),
        pltpu.SemaphoreType.DMA,
        pltpu.SemaphoreType.DMA,
    ],
    compiler_params=pltpu.CompilerParams(needs_layout_passes=False),
)


@jax.jit
def kernel(projections, scores):
    th = pl.pallas_call(
        _thresh_body,
        out_specs=pl.BlockSpec(memory_space=pltpu.SMEM),
        out_shape=jax.ShapeDtypeStruct((2,), jnp.float32),
    )(scores.reshape(128, 128))

    anchors, cnt = _sc_gather(projections, scores, jnp.pad(th, (0, 14)))

    pb16 = pl.pallas_call(
        _cast_body,
        grid=(8,),
        in_specs=[pl.BlockSpec((_N // 8, _D), lambda i: (i, 0))],
        out_specs=pl.BlockSpec((_N // 8, _D), lambda i: (i, 0)),
        out_shape=jax.ShapeDtypeStruct((_N, _D), jnp.bfloat16),
    )(projections)

    ab16 = pl.pallas_call(
        _cast_scale_body,
        grid=(8,),
        in_specs=[pl.BlockSpec((_AP // 8, _D), lambda i: (i, 0))],
        out_specs=pl.BlockSpec((_AP // 8, _D), lambda i: (i, 0)),
        out_shape=jax.ShapeDtypeStruct((_AP, _D), jnp.bfloat16),
    )(anchors)

    acc = pl.pallas_call(
        _main_body,
        grid=(_NR, _NC),
        in_specs=[
            pl.BlockSpec(memory_space=pltpu.SMEM),
            pl.BlockSpec((_R, _D), lambda i, j: (i, 0)),
            pl.BlockSpec((_C, _D), lambda i, j: (j, 0)),
            pl.BlockSpec((1, 1, _C), lambda i, j: (j, 0, 0)),
        ],
        out_specs=pl.BlockSpec((_R, 2), lambda i, j: (i, 0)),
        out_shape=jax.ShapeDtypeStruct((_AP, 2), jnp.float32),
        compiler_params=pltpu.CompilerParams(
            dimension_semantics=("arbitrary", "arbitrary")),
    )(
        th,
        ab16,
        pb16,
        scores.reshape(_NC, 1, _C),
    )

    loss = pl.pallas_call(
        _fin_body,
        in_specs=[
            pl.BlockSpec(memory_space=pltpu.SMEM),
            pl.BlockSpec((_AP, 2), lambda: (0, 0)),
            pl.BlockSpec((_AP, _D), lambda: (0, 0)),
        ],
        out_specs=pl.BlockSpec(memory_space=pltpu.SMEM),
        out_shape=jax.ShapeDtypeStruct((1,), jnp.float32),
    )(cnt, acc, anchors)
    return loss[0]


# speedup vs baseline: 1.5911x; 1.0086x over previous
import functools

import jax
import jax.numpy as jnp
from jax import lax
from jax.experimental import pallas as pl
from jax.experimental.pallas import tpu as pltpu
from jax.experimental.pallas import tpu_sc as plsc

_N = 16384
_D = 256
_INV_T = 10.0
_K_TOP = 6553
_K_BOT = 9832

_NW = 32
_AP = 6656
_WROWS = _AP // _NW

_R = 1664
_C = 2048
_NR = _AP // _R
_NC = _N // _C

_C1 = 14.426950408889634


def _cast_scale_body(x_ref, o_ref):
    o_ref[...] = (x_ref[...] * _C1).astype(jnp.bfloat16)


def _cast_body(x_ref, o_ref):
    o_ref[...] = x_ref[...].astype(jnp.bfloat16)


def _thresh_body(s_ref, out_ref):
    bits = lax.bitcast_convert_type(s_ref[...], jnp.int32)

    def step(_, carry):
        lo1, hi1, lo2, hi2 = carry

        def upd(lo, hi, k):
            mid = (lo + hi + 1) >> 1
            cnt = jnp.sum((bits >= mid).astype(jnp.int32))
            take = cnt >= k
            return jnp.where(take, mid, lo), jnp.where(take, hi, mid - 1)

        lo1, hi1 = upd(lo1, hi1, _K_TOP)
        lo2, hi2 = upd(lo2, hi2, _K_BOT)
        return lo1, hi1, lo2, hi2

    lo1, _, lo2, _ = lax.fori_loop(
        0, 30, step, (0, 0x3F800000, 0, 0x3F800000))
    out_ref[0] = lax.bitcast_convert_type(lo1, jnp.float32)
    out_ref[1] = lax.bitcast_convert_type(lo2, jnp.float32)


def _sc_body(proj_hbm, scores_hbm, th_hbm, anchors_hbm, cnt_hbm,
             s_v, th_v, idx_v, rows_a, rows_b, cnt_v, sem_a, sem_b):
    wid = lax.axis_index("s") * 2 + lax.axis_index("c")
    base = wid * _WROWS

    pltpu.sync_copy(scores_hbm, s_v)
    pltpu.sync_copy(th_hbm, th_v)
    lane = lax.iota(jnp.int32, 16)
    t_bot = th_v[...][1]

    zero16 = jnp.zeros((16,), jnp.int32)
    for k in range(_WROWS // 16):
        idx_v[pl.ds(base + k * 16, 16)] = zero16

    def body(it, off):
        s = s_v[pl.ds(it * 16, 16)]
        sel = s <= t_bot
        idxvec = lane + it * 16
        plsc.store_compressed(idx_v.at[pl.ds(off, 16)], idxvec, mask=sel)
        return off + jnp.sum(jnp.where(sel, 1, 0))

    total = lax.fori_loop(0, _N // 16, body, 0)

    @pl.when(wid == 0)
    def _write_count():
        cnt_v[...] = zero16 + total
        pltpu.sync_copy(cnt_v, cnt_hbm)

    h = _WROWS // 2
    cp_a = pltpu.async_copy(
        proj_hbm.at[idx_v.at[pl.ds(base, h)]], rows_a, sem_a)
    cp_b = pltpu.async_copy(
        proj_hbm.at[idx_v.at[pl.ds(base + h, h)]], rows_b, sem_b)
    cp_a.wait()
    pltpu.sync_copy(rows_a, anchors_hbm.at[pl.ds(base, h)])
    cp_b.wait()
    pltpu.sync_copy(rows_b, anchors_hbm.at[pl.ds(base + h, h)])


def _main_body(th_ref, a_ref, b_ref, scol_ref, out_ref):
    j = pl.program_id(1)
    t_top = th_ref[0]

    s = lax.dot_general(a_ref[...], b_ref[...], (((1,), (1,)), ((), ())),
                        preferred_element_type=jnp.float32)
    e = jnp.exp2(s - _C1).astype(jnp.bfloat16)
    m = (scol_ref[0, 0] >= t_top).astype(jnp.float32)
    red = jnp.concatenate(
        [m[:, None], jnp.ones((_C, 1), jnp.float32)], axis=1)
    p = lax.dot_general(e, red.astype(jnp.bfloat16),
                        (((1,), (0,)), ((), ())),
                        preferred_element_type=jnp.float32)
    out_ref[...] = jnp.where(j == 0, p, out_ref[...] + p)


def _fin_body(cnt_ref, acc_ref, a_ref, out_ref):
    cnt = jnp.minimum(cnt_ref[0], _AP)
    a32 = a_ref[...]
    rn = jnp.sum(a32 * a32, axis=1, keepdims=True) * _INV_T
    self_e = jnp.exp(rn - _INV_T)
    accn = acc_ref[:, 0:1]
    accd = acc_ref[:, 1:2] - self_e
    row_ids = lax.broadcasted_iota(jnp.int32, (_AP, 1), 0)
    rm = (row_ids < cnt).astype(jnp.float32)
    vals = (jnp.log(accd) - jnp.log(accn)) * rm
    out_ref[0] = jnp.sum(vals) / cnt.astype(jnp.float32)


_sc_gather = pl.kernel(
    _sc_body,
    out_type=[
        jax.ShapeDtypeStruct((_AP, _D), jnp.float32),
        jax.ShapeDtypeStruct((16,), jnp.int32),
    ],
    mesh=plsc.VectorSubcoreMesh(core_axis_name="c", subcore_axis_name="s",
                                num_cores=2, num_subcores=16),
    scratch_types=[
        pltpu.VMEM((_N,), jnp.float32),
        pltpu.VMEM((16,), jnp.float32),
        pltpu.VMEM((_AP + 16,), jnp.int32),
        pltpu.VMEM((_WROWS // 2, _D), jnp.float32),
        pltpu.VMEM((_WROWS // 2, _D), jnp.float32),
        pltpu.VMEM((16,), jnp.int32),
        pltpu.SemaphoreType.DMA,
        pltpu.SemaphoreType.DMA,
    ],
    compiler_params=pltpu.CompilerParams(needs_layout_passes=False),
)


@jax.jit
def kernel(projections, scores):
    th = pl.pallas_call(
        _thresh_body,
        out_specs=pl.BlockSpec(memory_space=pltpu.SMEM),
        out_shape=jax.ShapeDtypeStruct((2,), jnp.float32),
    )(scores.reshape(128, 128))

    anchors, cnt = _sc_gather(projections, scores, jnp.pad(th, (0, 14)))

    pb16 = pl.pallas_call(
        _cast_body,
        grid=(8,),
        in_specs=[pl.BlockSpec((_N // 8, _D), lambda i: (i, 0))],
        out_specs=pl.BlockSpec((_N // 8, _D), lambda i: (i, 0)),
        out_shape=jax.ShapeDtypeStruct((_N, _D), jnp.bfloat16),
    )(projections)

    ab16 = pl.pallas_call(
        _cast_scale_body,
        grid=(8,),
        in_specs=[pl.BlockSpec((_AP // 8, _D), lambda i: (i, 0))],
        out_specs=pl.BlockSpec((_AP // 8, _D), lambda i: (i, 0)),
        out_shape=jax.ShapeDtypeStruct((_AP, _D), jnp.bfloat16),
    )(anchors)

    acc = pl.pallas_call(
        _main_body,
        grid=(_NR, _NC),
        in_specs=[
            pl.BlockSpec(memory_space=pltpu.SMEM),
            pl.BlockSpec((_R, _D), lambda i, j: (i, 0)),
            pl.BlockSpec((_C, _D), lambda i, j: (j, 0)),
            pl.BlockSpec((1, 1, _C), lambda i, j: (j, 0, 0)),
        ],
        out_specs=pl.BlockSpec((_R, 2), lambda i, j: (i, 0)),
        out_shape=jax.ShapeDtypeStruct((_AP, 2), jnp.float32),
        compiler_params=pltpu.CompilerParams(
            dimension_semantics=("arbitrary", "arbitrary")),
    )(
        th,
        ab16,
        pb16,
        scores.reshape(_NC, 1, _C),
    )

    loss = pl.pallas_call(
        _fin_body,
        in_specs=[
            pl.BlockSpec(memory_space=pltpu.SMEM),
            pl.BlockSpec((_AP, 2), lambda: (0, 0)),
            pl.BlockSpec((_AP, _D), lambda: (0, 0)),
        ],
        out_specs=pl.BlockSpec(memory_space=pltpu.SMEM),
        out_shape=jax.ShapeDtypeStruct((1,), jnp.float32),
    )(cnt, acc, anchors)
    return loss[0]
